# Initial kernel scaffold; baseline (speedup 1.0000x reference)
import jax
import jax.numpy as jnp
from jax.experimental import pallas as pl


def kernel(x, w_pad, b_pad):
    raise NotImplementedError("write your pallas kernel here")


# speedup vs baseline: 4.1159x; 4.1159x over previous
import jax
import jax.numpy as jnp
from jax.experimental import pallas as pl
from jax.experimental.pallas import tpu as pltpu

N_OUT = 10
N_PAD = 128


def _round_up(x, m):
    return ((x + m - 1) // m) * m


def _readout_body(x_ref, w_ref, b_ref, o_ref):
    acc = jnp.dot(x_ref[...], w_ref[...], preferred_element_type=jnp.float32)
    o_ref[...] = (acc + b_ref[...])[:, :N_OUT].astype(o_ref.dtype)


def kernel(x, w_pad, b_pad):
    B = x.shape[0]
    x_flat = x.reshape(B, -1)
    units = x_flat.shape[1]
    kp = w_pad.shape[0]
    assert kp >= units and w_pad.shape[1] == N_PAD
    assert b_pad.shape == (1, N_PAD)

    if kp != units:
        x_flat = jnp.pad(x_flat, ((0, 0), (0, kp - units)))

    TB = min(512, _round_up(B, 8))
    Bp = _round_up(B, TB)
    if Bp != B:
        x_flat = jnp.pad(x_flat, ((0, Bp - B), (0, 0)))

    itemsize = jnp.dtype(x_flat.dtype).itemsize
    cost = pl.CostEstimate(
        flops=2 * Bp * kp * N_PAD,
        transcendentals=0,
        bytes_accessed=(Bp * kp + kp * N_PAD + Bp * N_OUT) * itemsize,
    )

    out = pl.pallas_call(
        _readout_body,
        out_shape=jax.ShapeDtypeStruct((Bp, N_OUT), x_flat.dtype),
        grid=(Bp // TB,),
        in_specs=[
            pl.BlockSpec((TB, kp), lambda i: (i, 0)),
            pl.BlockSpec((kp, N_PAD), lambda i: (0, 0)),
            pl.BlockSpec((1, N_PAD), lambda i: (0, 0)),
        ],
        out_specs=pl.BlockSpec((TB, N_OUT), lambda i: (i, 0)),
        compiler_params=pltpu.CompilerParams(
            dimension_semantics=("parallel",),
        ),
        cost_estimate=cost,
    )(x_flat, w_pad, b_pad)

    return out[:B]
